# baseline (speedup 1.0000x reference)
import functools

import jax
import jax.numpy as jnp
from jax import lax
from jax.experimental import pallas as pl
from jax.experimental.pallas import tpu as pltpu
from jax.experimental.pallas import tpu_sc as plsc

N = 10000
E = 320000
F = 128
G = 32
CUTOFF = 5.0
NCONV = 3

NC, NS, L = 2, 16, 16
NW = NC * NS
NP = 10112
EW = E // NW
CK = 32
KB = 16
KBP = 16
NB = 20
NSB = NB // 2
NCH = KB * NB
EWP = NCH * CK
EP = NW * EWP
RPT = NP // NS
_RCHUNKS = [2 * CK] * (RPT // (2 * CK)) + ([RPT % (2 * CK)] if RPT % (2 * CK) else [])

_LN2 = 0.6931471805599453


def _ssp(x):
    return jnp.maximum(x, 0.0) + jnp.log1p(jnp.exp(-jnp.abs(x))) - _LN2


def _pack_pairs(x):
    u = lax.bitcast_convert_type(x, jnp.uint32)
    r = (u + jnp.uint32(0x7FFF) + ((u >> 16) & jnp.uint32(1))) >> 16
    w = r[:, :64] | (r[:, 64:] << 16)
    return lax.bitcast_convert_type(w, jnp.int32)


def _mesh():
    return plsc.VectorSubcoreMesh(
        core_axis_name="c", subcore_axis_name="s", num_cores=NC, num_subcores=NS
    )


def _sc_e2_body(xc, yc, zc, srcf, dstf, out, xt, yt, zt, sb, db, e2b):
    wid = lax.axis_index("s") * NC + lax.axis_index("c")
    base = wid * EWP
    pltpu.sync_copy(xc, xt)
    pltpu.sync_copy(yc, yt)
    pltpu.sync_copy(zc, zt)
    pltpu.sync_copy(srcf.at[pl.ds(base, EWP)], sb)
    pltpu.sync_copy(dstf.at[pl.ds(base, EWP)], db)

    def step(t, carry):
        sl = pl.ds(t * L, L)
        sv = sb[sl]
        dv = db[sl]
        dx = plsc.load_gather(xt, [sv]) - plsc.load_gather(xt, [dv])
        dy = plsc.load_gather(yt, [sv]) - plsc.load_gather(yt, [dv])
        dz = plsc.load_gather(zt, [sv]) - plsc.load_gather(zt, [dv])
        e2b[sl] = dx * dx + dy * dy + dz * dz
        return carry

    lax.fori_loop(0, EWP // L, step, None)
    pltpu.sync_copy(e2b, out.at[pl.ds(base, EWP)])


def _sc_e2(xc, yc, zc, srcf, dstf):
    f = pl.kernel(
        _sc_e2_body,
        out_type=jax.ShapeDtypeStruct((EP,), jnp.float32),
        mesh=_mesh(),
        scratch_types=[
            pltpu.VMEM((NP,), jnp.float32),
            pltpu.VMEM((NP,), jnp.float32),
            pltpu.VMEM((NP,), jnp.float32),
            pltpu.VMEM((EWP,), jnp.int32),
            pltpu.VMEM((EWP,), jnp.int32),
            pltpu.VMEM((EWP,), jnp.float32),
        ],
        compiler_params=pltpu.CompilerParams(needs_layout_passes=False),
    )
    return f(xc, yc, zc, srcf, dstf)


def _sc_msg_body(rn, eh, idxg, idxs, out, aggs, gA, gB, sA, sB,
                 gb0, gb1, eb0, eb1, pr0, pr1,
                 sg0, sg1, se0, se1, ss0, ss1, six):
    core = lax.axis_index("c")
    sub = lax.axis_index("s")
    wid = sub * NC + core
    base = wid * EWP
    gb = (gb0, gb1)
    eb = (eb0, eb1)
    pr = (pr0, pr1)
    sg = (sg0, sg1)
    se = (se0, se1)
    ss = (ss0, ss1)

    def gidx(buf, k):
        return buf.at[k, pl.ds(0, 2 * CK)]

    def sidx(buf, k):
        return buf.at[k, pl.ds(0, 2 * CK)]

    def ib_row(m):
        return (gA, sA, m) if m < KB else (gB, sB, m - KB)

    def zrow(r, carry):
        for q in range(F // L):
            pr0[r, pl.ds(q * L, L)] = jnp.zeros((L,), jnp.float32)
        return carry

    lax.fori_loop(0, 2 * CK, zrow, None)
    row0 = sub * RPT
    off = 0
    for sz in _RCHUNKS:
        pltpu.sync_copy(pr0.at[pl.ds(0, sz)], aggs.at[pl.ds(row0 + off, sz)])
        off += sz
    plsc.subcore_barrier()

    pltpu.sync_copy(idxg.at[wid, 0], gA)
    pltpu.sync_copy(idxs.at[wid, 0], sA)
    pltpu.async_copy(idxg.at[wid, 1], gB, six)
    pltpu.async_copy(idxs.at[wid, 1], sB, six)
    pltpu.async_copy(rn.at[gidx(gA, 0)], gb0, sg0)
    pltpu.async_copy(eh.at[pl.ds(base, CK)], eb0, se0)

    def sbatch(t, carry):
        jb = t * 2 * KB
        for m in range(2 * KB):
            p = m & 1
            q = 1 - p
            j = jb + m
            last_slot = m == 2 * KB - 1
            if m == KB - 1:
                pltpu.make_async_copy(idxg.at[wid, 2 * t + 1], gB, six).wait()
                pltpu.make_async_copy(idxs.at[wid, 2 * t + 1], sB, six).wait()
            ngbuf, _, nrow = ib_row(m + 1) if not last_slot else (gA, sA, 0)
            if last_slot:
                @pl.when(t < NSB - 1)
                def _():
                    pltpu.make_async_copy(idxg.at[wid, 2 * t + 2], gA, six).wait()
                    pltpu.make_async_copy(idxs.at[wid, 2 * t + 2], sA, six).wait()
                    pltpu.async_copy(rn.at[gidx(gA, 0)], gb[q], sg[q])
                    pltpu.async_copy(eh.at[pl.ds(base + (j + 1) * CK, CK)],
                                     eb[q], se[q])
            else:
                pltpu.async_copy(rn.at[gidx(ngbuf, nrow)], gb[q], sg[q])
                pltpu.async_copy(eh.at[pl.ds(base + (j + 1) * CK, CK)],
                                 eb[q], se[q])
            pltpu.make_async_copy(rn.at[gidx(gA, 0)], gb[p], sg[p]).wait()
            pltpu.make_async_copy(eh.at[pl.ds(base, CK)], eb[p], se[p]).wait()
            if m < 2:
                @pl.when(t > 0)
                def _():
                    pltpu.make_async_copy(pr[p], aggs.at[sidx(sA, 0)], ss[p]).wait()
            else:
                pltpu.make_async_copy(pr[p], aggs.at[sidx(sA, 0)], ss[p]).wait()
            if m == 1:
                @pl.when(t > 0)
                def _():
                    pltpu.async_copy(idxg.at[wid, 2 * t + 1], gB, six)
                    pltpu.async_copy(idxs.at[wid, 2 * t + 1], sB, six)
            if m == KB + 1:
                @pl.when(t < NSB - 1)
                def _():
                    pltpu.async_copy(idxg.at[wid, 2 * t + 2], gA, six)
                    pltpu.async_copy(idxs.at[wid, 2 * t + 2], sA, six)
            gbp = gb[p]
            ebp = eb[p]
            prp = pr[p]

            def mrow(r, carry3):
                for q4 in range(4):
                    slw = pl.ds(q4 * L, L)
                    lo = pl.ds(q4 * L, L)
                    hi = pl.ds(64 + q4 * L, L)
                    ea, ebv = plsc.unpack(
                        plsc.bitcast(ebp[r, slw], jnp.bfloat16),
                        format=plsc.PackFormat.INTERLEAVED,
                        preferred_element_type=jnp.float32)
                    prp[r, lo] = gbp[r, lo] * ea
                    prp[r, hi] = gbp[r, hi] * ebv
                    prp[CK + r, lo] = gbp[CK + r, lo] * ea
                    prp[CK + r, hi] = gbp[CK + r, hi] * ebv
                return carry3

            lax.fori_loop(0, CK, mrow, None)
            _, csbuf, crow = ib_row(m)
            pltpu.async_copy(prp, aggs.at[sidx(csbuf, crow)], ss[p], add=True)
        return carry

    lax.fori_loop(0, NSB, sbatch, None)
    pltpu.make_async_copy(pr0, aggs.at[sidx(sA, 0)], ss0).wait()
    pltpu.make_async_copy(pr1, aggs.at[sidx(sA, 0)], ss1).wait()
    plsc.subcore_barrier()

    off = 0
    for sz in _RCHUNKS:
        rows = pl.ds(row0 + off, sz)
        pltpu.sync_copy(aggs.at[rows], pr0.at[pl.ds(0, sz)])
        pltpu.sync_copy(pr0.at[pl.ds(0, sz)], out.at[core, rows])
        off += sz


def _sc_msg(rn, eh, idxg, idxs):
    f = pl.kernel(
        _sc_msg_body,
        out_type=jax.ShapeDtypeStruct((NC, NP, F), jnp.float32),
        mesh=_mesh(),
        scratch_types=[
            pltpu.VMEM_SHARED((NP, F), jnp.float32),
            pltpu.VMEM((KBP, 2 * CK), jnp.int32),
            pltpu.VMEM((KBP, 2 * CK), jnp.int32),
            pltpu.VMEM((KBP, 2 * CK), jnp.int32),
            pltpu.VMEM((KBP, 2 * CK), jnp.int32),
            pltpu.VMEM((2 * CK, F), jnp.float32),
            pltpu.VMEM((2 * CK, F), jnp.float32),
            pltpu.VMEM((CK, F // 2), jnp.int32),
            pltpu.VMEM((CK, F // 2), jnp.int32),
            pltpu.VMEM((2 * CK, F), jnp.float32),
            pltpu.VMEM((2 * CK, F), jnp.float32),
            pltpu.SemaphoreType.DMA,
            pltpu.SemaphoreType.DMA,
            pltpu.SemaphoreType.DMA,
            pltpu.SemaphoreType.DMA,
            pltpu.SemaphoreType.DMA,
            pltpu.SemaphoreType.DMA,
            pltpu.SemaphoreType.DMA,
        ],
        compiler_params=pltpu.CompilerParams(needs_layout_passes=False),
    )
    return f(rn, eh, idxg, idxs)


_BLK = NP // 16
_NBLK = 16


def _emb_body(z_ref, emb_ref, o_ref):
    zb = z_ref[...]
    oh = (zb == lax.broadcasted_iota(jnp.int32, (_BLK, 100), 1)).astype(jnp.float32)
    o_ref[...] = jnp.dot(oh, emb_ref[...], preferred_element_type=jnp.float32)


def _tc_emb(zcol, emb):
    return pl.pallas_call(
        _emb_body,
        grid=(_NBLK,),
        in_specs=[
            pl.BlockSpec((_BLK, 1), lambda i: (i, 0)),
            pl.BlockSpec((100, F), lambda i: (0, 0)),
        ],
        out_specs=pl.BlockSpec((_BLK, F), lambda i: (i, 0)),
        out_shape=jax.ShapeDtypeStruct((NP, F), jnp.float32),
    )(zcol, emb)


_EBLK = 512


def _eh_body(e2_ref, w1_ref, b1_ref, w2_ref, b2_ref, o1):
    e = jnp.sqrt(jnp.maximum(e2_ref[...], 0.0))
    width = CUTOFF / (G - 1)
    coeff = -0.5 / (width * width)
    off = lax.broadcasted_iota(jnp.int32, (1, G), 1).astype(jnp.float32) * width
    d = e - off
    g = jnp.exp(coeff * d * d)
    h = _ssp(jnp.dot(g, w1_ref[...], preferred_element_type=jnp.float32)
             + b1_ref[...])
    o1[...] = _pack_pairs(jnp.dot(h, w2_ref[...], preferred_element_type=jnp.float32)
                          + b2_ref[...])


def _tc_eh(e2col, w1, b1, w2, b2):
    return pl.pallas_call(
        _eh_body,
        grid=(EP // _EBLK,),
        in_specs=[
            pl.BlockSpec((_EBLK, 1), lambda i: (i, 0)),
            pl.BlockSpec((G, G), lambda i: (0, 0)),
            pl.BlockSpec((1, G), lambda i: (0, 0)),
            pl.BlockSpec((G, F), lambda i: (0, 0)),
            pl.BlockSpec((1, F), lambda i: (0, 0)),
        ],
        out_specs=pl.BlockSpec((_EBLK, F // 2), lambda i: (i, 0)),
        out_shape=jax.ShapeDtypeStruct((EP, F // 2), jnp.int32),
    )(e2col, w1, b1, w2, b2)


_RBLK = NP // 8


def _rn_body(r_ref, w_ref, b_ref, o_ref):
    o_ref[...] = (jnp.dot(r_ref[...], w_ref[...], preferred_element_type=jnp.float32)
                  + b_ref[...])


def _tc_rn(r, w, b):
    return pl.pallas_call(
        _rn_body,
        grid=(8,),
        in_specs=[
            pl.BlockSpec((_RBLK, F), lambda i: (i, 0)),
            pl.BlockSpec((F, F), lambda i: (0, 0)),
            pl.BlockSpec((1, F), lambda i: (0, 0)),
        ],
        out_specs=pl.BlockSpec((_RBLK, F), lambda i: (i, 0)),
        out_shape=jax.ShapeDtypeStruct((NP, F), jnp.float32),
    )(r, w, b)


def _upd_body(r_ref, agg_ref, w1_ref, b1_ref, w2_ref, b2_ref, o_ref):
    a = agg_ref[...][0] + agg_ref[...][1]
    h = _ssp(jnp.dot(a, w1_ref[...], preferred_element_type=jnp.float32) + b1_ref[...])
    o_ref[...] = (r_ref[...]
                  + jnp.dot(h, w2_ref[...], preferred_element_type=jnp.float32)
                  + b2_ref[...])


def _tc_update(r, aggp, w1, b1, w2, b2):
    return pl.pallas_call(
        _upd_body,
        grid=(_NBLK,),
        in_specs=[
            pl.BlockSpec((_BLK, F), lambda i: (i, 0)),
            pl.BlockSpec((NC, _BLK, F), lambda i: (0, i, 0)),
            pl.BlockSpec((F, F), lambda i: (0, 0)),
            pl.BlockSpec((1, F), lambda i: (0, 0)),
            pl.BlockSpec((F, F), lambda i: (0, 0)),
            pl.BlockSpec((1, F), lambda i: (0, 0)),
        ],
        out_specs=pl.BlockSpec((_BLK, F), lambda i: (i, 0)),
        out_shape=jax.ShapeDtypeStruct((NP, F), jnp.float32),
    )(r, aggp, w1, b1, w2, b2)


def _ro_body(r_ref, w1_ref, b1_ref, w2_ref, b2_ref, o_ref):
    h = _ssp(jnp.dot(r_ref[...], w1_ref[...], preferred_element_type=jnp.float32)
             + b1_ref[...])
    o_ref[...] = jnp.dot(h, w2_ref[...], preferred_element_type=jnp.float32) + b2_ref[...]


def _tc_readout(r, w1, b1, w2, b2):
    return pl.pallas_call(
        _ro_body,
        grid=(_NBLK,),
        in_specs=[
            pl.BlockSpec((_BLK, F), lambda i: (i, 0)),
            pl.BlockSpec((F, F // 2), lambda i: (0, 0)),
            pl.BlockSpec((1, F // 2), lambda i: (0, 0)),
            pl.BlockSpec((F // 2, 1), lambda i: (0, 0)),
            pl.BlockSpec((1, 1), lambda i: (0, 0)),
        ],
        out_specs=pl.BlockSpec((_BLK, 1), lambda i: (i, 0)),
        out_shape=jax.ShapeDtypeStruct((NP, 1), jnp.float32),
    )(r, w1, b1, w2, b2)


def kernel(nxyz, nbr_list, num_atoms, params):
    xyz = nxyz[:, 1:4]
    z = nxyz[:, 0].astype(jnp.int32)
    src = nbr_list[:, 0]
    dst = nbr_list[:, 1]

    pad = ((0, 0), (0, EWP - EW))
    srcw = jnp.pad(src.reshape(NW, EW), pad, constant_values=N)
    dstw = jnp.pad(dst.reshape(NW, EW), pad, constant_values=N)
    sc3 = srcw.reshape(NW, NCH, CK)
    dc3 = dstw.reshape(NW, NCH, CK)
    kpad = ((0, 0), (0, 0), (0, KBP - KB), (0, 0))
    idxg = jnp.pad(jnp.concatenate([sc3, dc3], axis=-1)
                   .reshape(NW, NB, KB, 2 * CK), kpad, constant_values=N)
    idxs = jnp.pad(jnp.concatenate([dc3, sc3], axis=-1)
                   .reshape(NW, NB, KB, 2 * CK), kpad, constant_values=N)

    cpad = jnp.zeros((3, NP), jnp.float32).at[:, :N].set(xyz.T)
    xc, yc, zc = cpad[0], cpad[1], cpad[2]
    zcol = jnp.zeros((NP, 1), jnp.int32).at[:N, 0].set(z)

    e2 = _sc_e2(xc, yc, zc, srcw.reshape(-1), dstw.reshape(-1))
    e2col = e2.reshape(EP, 1)

    cv = params["convs"]
    ehs = [_tc_eh(e2col, c["edge1"]["w"], c["edge1"]["b"][None, :],
                  c["edge2"]["w"], c["edge2"]["b"][None, :]) for c in cv]

    r = _tc_emb(zcol, params["emb"])
    for c in range(NCONV):
        cp = cv[c]
        rn = _tc_rn(r, cp["node"]["w"], cp["node"]["b"][None, :])
        aggp = _sc_msg(rn, ehs[c], idxg, idxs)
        r = _tc_update(r, aggp, cp["upd1"]["w"], cp["upd1"]["b"][None, :],
                       cp["upd2"]["w"], cp["upd2"]["b"][None, :])

    ro = params["readout"]
    h = _tc_readout(r, ro["l1"]["w"], ro["l1"]["b"][None, :],
                    ro["l2"]["w"], ro["l2"]["b"][None, :])
    return h[:N]


# speedup vs baseline: 1.4402x; 1.4402x over previous
import functools

import jax
import jax.numpy as jnp
from jax import lax
from jax.experimental import pallas as pl
from jax.experimental.pallas import tpu as pltpu
from jax.experimental.pallas import tpu_sc as plsc

N = 10000
E = 320000
F = 128
G = 32
CUTOFF = 5.0
NCONV = 3

NC, NS, L = 2, 16, 16
NW = NC * NS
NP = 10112
EW = E // NW
CK = 48
KB = 15
KBP = 16
NB = 14
NSB = NB // 2
NCH = KB * NB
EWP = NCH * CK
EP = NW * EWP
RPT = NP // NS
_RCHUNKS = [2 * CK] * (RPT // (2 * CK)) + ([RPT % (2 * CK)] if RPT % (2 * CK) else [])

_LN2 = 0.6931471805599453


def _ssp(x):
    return jnp.maximum(x, 0.0) + jnp.log1p(jnp.exp(-jnp.abs(x))) - _LN2


def _pack_pairs(x):
    u = lax.bitcast_convert_type(x, jnp.uint32)
    r = (u + jnp.uint32(0x7FFF) + ((u >> 16) & jnp.uint32(1))) >> 16
    w = r[:, :64] | (r[:, 64:] << 16)
    return lax.bitcast_convert_type(w, jnp.int32)


def _mesh():
    return plsc.VectorSubcoreMesh(
        core_axis_name="c", subcore_axis_name="s", num_cores=NC, num_subcores=NS
    )


def _sc_e2_body(xc, yc, zc, srcf, dstf, out, xt, yt, zt, sb, db, e2b):
    wid = lax.axis_index("s") * NC + lax.axis_index("c")
    base = wid * EWP
    pltpu.sync_copy(xc, xt)
    pltpu.sync_copy(yc, yt)
    pltpu.sync_copy(zc, zt)
    pltpu.sync_copy(srcf.at[pl.ds(base, EWP)], sb)
    pltpu.sync_copy(dstf.at[pl.ds(base, EWP)], db)

    def step(t, carry):
        sl = pl.ds(t * L, L)
        sv = sb[sl]
        dv = db[sl]
        dx = plsc.load_gather(xt, [sv]) - plsc.load_gather(xt, [dv])
        dy = plsc.load_gather(yt, [sv]) - plsc.load_gather(yt, [dv])
        dz = plsc.load_gather(zt, [sv]) - plsc.load_gather(zt, [dv])
        e2b[sl] = dx * dx + dy * dy + dz * dz
        return carry

    lax.fori_loop(0, EWP // L, step, None)
    pltpu.sync_copy(e2b, out.at[pl.ds(base, EWP)])


def _sc_e2(xc, yc, zc, srcf, dstf):
    f = pl.kernel(
        _sc_e2_body,
        out_type=jax.ShapeDtypeStruct((EP,), jnp.float32),
        mesh=_mesh(),
        scratch_types=[
            pltpu.VMEM((NP,), jnp.float32),
            pltpu.VMEM((NP,), jnp.float32),
            pltpu.VMEM((NP,), jnp.float32),
            pltpu.VMEM((EWP,), jnp.int32),
            pltpu.VMEM((EWP,), jnp.int32),
            pltpu.VMEM((EWP,), jnp.float32),
        ],
        compiler_params=pltpu.CompilerParams(needs_layout_passes=False),
    )
    return f(xc, yc, zc, srcf, dstf)


def _sc_msg_body(rn, eh, idxg, idxs, out, aggs, gA, gB, sA, sB,
                 gb0, gb1, eb0, eb1,
                 sg0, sg1, se0, se1, ss0, ss1, six):
    core = lax.axis_index("c")
    sub = lax.axis_index("s")
    wid = sub * NC + core
    base = wid * EWP
    gb = (gb0, gb1)
    eb = (eb0, eb1)
    sg = (sg0, sg1)
    se = (se0, se1)
    ss = (ss0, ss1)

    def gidx(buf, k):
        return buf.at[k, pl.ds(0, 2 * CK)]

    def sidx(buf, k):
        return buf.at[k, pl.ds(0, 2 * CK)]

    def ib_row(m):
        return (gA, sA, m) if m < KB else (gB, sB, m - KB)

    def zrow(r, carry):
        for q in range(F // L):
            gb0[r, pl.ds(q * L, L)] = jnp.zeros((L,), jnp.float32)
        return carry

    lax.fori_loop(0, 2 * CK, zrow, None)
    row0 = sub * RPT
    off = 0
    for sz in _RCHUNKS:
        pltpu.sync_copy(gb0.at[pl.ds(0, sz)], aggs.at[pl.ds(row0 + off, sz)])
        off += sz
    plsc.subcore_barrier()

    pltpu.sync_copy(idxg.at[wid, 0], gA)
    pltpu.sync_copy(idxs.at[wid, 0], sA)
    pltpu.async_copy(idxg.at[wid, 1], gB, six)
    pltpu.async_copy(idxs.at[wid, 1], sB, six)
    pltpu.async_copy(rn.at[gidx(gA, 0)], gb0, sg0)
    pltpu.async_copy(eh.at[pl.ds(base, CK)], eb0, se0)

    def sbatch(t, carry):
        jb = t * 2 * KB
        for m in range(2 * KB):
            p = m & 1
            q = 1 - p
            j = jb + m
            last_slot = m == 2 * KB - 1
            if m == 0:
                @pl.when(t > 0)
                def _():
                    pltpu.make_async_copy(gb[q], aggs.at[sidx(sA, 0)], ss[q]).wait()
                    pltpu.async_copy(idxg.at[wid, 2 * t + 1], gB, six)
                    pltpu.async_copy(idxs.at[wid, 2 * t + 1], sB, six)
            else:
                pltpu.make_async_copy(gb[q], aggs.at[sidx(sA, 0)], ss[q]).wait()
            if m == KB - 1:
                pltpu.make_async_copy(idxg.at[wid, 2 * t + 1], gB, six).wait()
                pltpu.make_async_copy(idxs.at[wid, 2 * t + 1], sB, six).wait()
            ngbuf, _, nrow = ib_row(m + 1) if not last_slot else (gA, sA, 0)
            if last_slot:
                @pl.when(t < NSB - 1)
                def _():
                    pltpu.make_async_copy(idxg.at[wid, 2 * t + 2], gA, six).wait()
                    pltpu.make_async_copy(idxs.at[wid, 2 * t + 2], sA, six).wait()
                    pltpu.async_copy(rn.at[gidx(gA, 0)], gb[q], sg[q])
                    pltpu.async_copy(eh.at[pl.ds(base + (j + 1) * CK, CK)],
                                     eb[q], se[q])
            else:
                pltpu.async_copy(rn.at[gidx(ngbuf, nrow)], gb[q], sg[q])
                pltpu.async_copy(eh.at[pl.ds(base + (j + 1) * CK, CK)],
                                 eb[q], se[q])
            pltpu.make_async_copy(rn.at[gidx(gA, 0)], gb[p], sg[p]).wait()
            pltpu.make_async_copy(eh.at[pl.ds(base, CK)], eb[p], se[p]).wait()
            if m == KB + 1:
                @pl.when(t < NSB - 1)
                def _():
                    pltpu.async_copy(idxg.at[wid, 2 * t + 2], gA, six)
                    pltpu.async_copy(idxs.at[wid, 2 * t + 2], sA, six)
            gbp = gb[p]
            ebp = eb[p]

            def mrow(r, carry3):
                for q4 in range(4):
                    slw = pl.ds(q4 * L, L)
                    lo = pl.ds(q4 * L, L)
                    hi = pl.ds(64 + q4 * L, L)
                    ea, ebv = plsc.unpack(
                        plsc.bitcast(ebp[r, slw], jnp.bfloat16),
                        format=plsc.PackFormat.INTERLEAVED,
                        preferred_element_type=jnp.float32)
                    gbp[r, lo] = gbp[r, lo] * ea
                    gbp[r, hi] = gbp[r, hi] * ebv
                    gbp[CK + r, lo] = gbp[CK + r, lo] * ea
                    gbp[CK + r, hi] = gbp[CK + r, hi] * ebv
                return carry3

            lax.fori_loop(0, CK, mrow, None)
            _, csbuf, crow = ib_row(m)
            pltpu.async_copy(gbp, aggs.at[sidx(csbuf, crow)], ss[p], add=True)
        return carry

    lax.fori_loop(0, NSB, sbatch, None)
    pltpu.make_async_copy(gb1, aggs.at[sidx(sA, 0)], ss1).wait()
    plsc.subcore_barrier()

    off = 0
    for sz in _RCHUNKS:
        rows = pl.ds(row0 + off, sz)
        pltpu.sync_copy(aggs.at[rows], gb0.at[pl.ds(0, sz)])
        pltpu.sync_copy(gb0.at[pl.ds(0, sz)], out.at[core, rows])
        off += sz


def _sc_msg(rn, eh, idxg, idxs):
    f = pl.kernel(
        _sc_msg_body,
        out_type=jax.ShapeDtypeStruct((NC, NP, F), jnp.float32),
        mesh=_mesh(),
        scratch_types=[
            pltpu.VMEM_SHARED((NP, F), jnp.float32),
            pltpu.VMEM((KBP, 2 * CK), jnp.int32),
            pltpu.VMEM((KBP, 2 * CK), jnp.int32),
            pltpu.VMEM((KBP, 2 * CK), jnp.int32),
            pltpu.VMEM((KBP, 2 * CK), jnp.int32),
            pltpu.VMEM((2 * CK, F), jnp.float32),
            pltpu.VMEM((2 * CK, F), jnp.float32),
            pltpu.VMEM((CK, F // 2), jnp.int32),
            pltpu.VMEM((CK, F // 2), jnp.int32),
            pltpu.SemaphoreType.DMA,
            pltpu.SemaphoreType.DMA,
            pltpu.SemaphoreType.DMA,
            pltpu.SemaphoreType.DMA,
            pltpu.SemaphoreType.DMA,
            pltpu.SemaphoreType.DMA,
            pltpu.SemaphoreType.DMA,
        ],
        compiler_params=pltpu.CompilerParams(needs_layout_passes=False),
    )
    return f(rn, eh, idxg, idxs)


_BLK = NP // 16
_NBLK = 16


def _emb_body(z_ref, emb_ref, o_ref):
    zb = z_ref[...]
    oh = (zb == lax.broadcasted_iota(jnp.int32, (_BLK, 100), 1)).astype(jnp.float32)
    o_ref[...] = jnp.dot(oh, emb_ref[...], preferred_element_type=jnp.float32)


def _tc_emb(zcol, emb):
    return pl.pallas_call(
        _emb_body,
        grid=(_NBLK,),
        in_specs=[
            pl.BlockSpec((_BLK, 1), lambda i: (i, 0)),
            pl.BlockSpec((100, F), lambda i: (0, 0)),
        ],
        out_specs=pl.BlockSpec((_BLK, F), lambda i: (i, 0)),
        out_shape=jax.ShapeDtypeStruct((NP, F), jnp.float32),
    )(zcol, emb)


_EBLK = 512


def _eh_body(e2_ref, w1_ref, b1_ref, w2_ref, b2_ref, o1):
    e = jnp.sqrt(jnp.maximum(e2_ref[...], 0.0))
    width = CUTOFF / (G - 1)
    coeff = -0.5 / (width * width)
    off = lax.broadcasted_iota(jnp.int32, (1, G), 1).astype(jnp.float32) * width
    d = e - off
    g = jnp.exp(coeff * d * d)
    h = _ssp(jnp.dot(g, w1_ref[...], preferred_element_type=jnp.float32)
             + b1_ref[...])
    o1[...] = _pack_pairs(jnp.dot(h, w2_ref[...], preferred_element_type=jnp.float32)
                          + b2_ref[...])


def _tc_eh(e2col, w1, b1, w2, b2):
    return pl.pallas_call(
        _eh_body,
        grid=(EP // _EBLK,),
        in_specs=[
            pl.BlockSpec((_EBLK, 1), lambda i: (i, 0)),
            pl.BlockSpec((G, G), lambda i: (0, 0)),
            pl.BlockSpec((1, G), lambda i: (0, 0)),
            pl.BlockSpec((G, F), lambda i: (0, 0)),
            pl.BlockSpec((1, F), lambda i: (0, 0)),
        ],
        out_specs=pl.BlockSpec((_EBLK, F // 2), lambda i: (i, 0)),
        out_shape=jax.ShapeDtypeStruct((EP, F // 2), jnp.int32),
    )(e2col, w1, b1, w2, b2)


_RBLK = NP // 8


def _rn_body(r_ref, w_ref, b_ref, o_ref):
    o_ref[...] = (jnp.dot(r_ref[...], w_ref[...], preferred_element_type=jnp.float32)
                  + b_ref[...])


def _tc_rn(r, w, b):
    return pl.pallas_call(
        _rn_body,
        grid=(8,),
        in_specs=[
            pl.BlockSpec((_RBLK, F), lambda i: (i, 0)),
            pl.BlockSpec((F, F), lambda i: (0, 0)),
            pl.BlockSpec((1, F), lambda i: (0, 0)),
        ],
        out_specs=pl.BlockSpec((_RBLK, F), lambda i: (i, 0)),
        out_shape=jax.ShapeDtypeStruct((NP, F), jnp.float32),
    )(r, w, b)


def _upd_body(r_ref, agg_ref, w1_ref, b1_ref, w2_ref, b2_ref, o_ref):
    a = agg_ref[...][0] + agg_ref[...][1]
    h = _ssp(jnp.dot(a, w1_ref[...], preferred_element_type=jnp.float32) + b1_ref[...])
    o_ref[...] = (r_ref[...]
                  + jnp.dot(h, w2_ref[...], preferred_element_type=jnp.float32)
                  + b2_ref[...])


def _tc_update(r, aggp, w1, b1, w2, b2):
    return pl.pallas_call(
        _upd_body,
        grid=(_NBLK,),
        in_specs=[
            pl.BlockSpec((_BLK, F), lambda i: (i, 0)),
            pl.BlockSpec((NC, _BLK, F), lambda i: (0, i, 0)),
            pl.BlockSpec((F, F), lambda i: (0, 0)),
            pl.BlockSpec((1, F), lambda i: (0, 0)),
            pl.BlockSpec((F, F), lambda i: (0, 0)),
            pl.BlockSpec((1, F), lambda i: (0, 0)),
        ],
        out_specs=pl.BlockSpec((_BLK, F), lambda i: (i, 0)),
        out_shape=jax.ShapeDtypeStruct((NP, F), jnp.float32),
    )(r, aggp, w1, b1, w2, b2)


def _ro_body(r_ref, w1_ref, b1_ref, w2_ref, b2_ref, o_ref):
    h = _ssp(jnp.dot(r_ref[...], w1_ref[...], preferred_element_type=jnp.float32)
             + b1_ref[...])
    o_ref[...] = jnp.dot(h, w2_ref[...], preferred_element_type=jnp.float32) + b2_ref[...]


def _tc_readout(r, w1, b1, w2, b2):
    return pl.pallas_call(
        _ro_body,
        grid=(_NBLK,),
        in_specs=[
            pl.BlockSpec((_BLK, F), lambda i: (i, 0)),
            pl.BlockSpec((F, F // 2), lambda i: (0, 0)),
            pl.BlockSpec((1, F // 2), lambda i: (0, 0)),
            pl.BlockSpec((F // 2, 1), lambda i: (0, 0)),
            pl.BlockSpec((1, 1), lambda i: (0, 0)),
        ],
        out_specs=pl.BlockSpec((_BLK, 1), lambda i: (i, 0)),
        out_shape=jax.ShapeDtypeStruct((NP, 1), jnp.float32),
    )(r, w1, b1, w2, b2)


def kernel(nxyz, nbr_list, num_atoms, params):
    xyz = nxyz[:, 1:4]
    z = nxyz[:, 0].astype(jnp.int32)
    src = nbr_list[:, 0]
    dst = nbr_list[:, 1]

    pad = ((0, 0), (0, EWP - EW))
    srcw = jnp.pad(src.reshape(NW, EW), pad, constant_values=N)
    dstw = jnp.pad(dst.reshape(NW, EW), pad, constant_values=N)
    sc3 = srcw.reshape(NW, NCH, CK)
    dc3 = dstw.reshape(NW, NCH, CK)
    kpad = ((0, 0), (0, 0), (0, KBP - KB), (0, 0))
    idxg = jnp.pad(jnp.concatenate([sc3, dc3], axis=-1)
                   .reshape(NW, NB, KB, 2 * CK), kpad, constant_values=N)
    idxs = jnp.pad(jnp.concatenate([dc3, sc3], axis=-1)
                   .reshape(NW, NB, KB, 2 * CK), kpad, constant_values=N)

    cpad = jnp.zeros((3, NP), jnp.float32).at[:, :N].set(xyz.T)
    xc, yc, zc = cpad[0], cpad[1], cpad[2]
    zcol = jnp.zeros((NP, 1), jnp.int32).at[:N, 0].set(z)

    e2 = _sc_e2(xc, yc, zc, srcw.reshape(-1), dstw.reshape(-1))
    e2col = e2.reshape(EP, 1)

    cv = params["convs"]
    ehs = [_tc_eh(e2col, c["edge1"]["w"], c["edge1"]["b"][None, :],
                  c["edge2"]["w"], c["edge2"]["b"][None, :]) for c in cv]

    r = _tc_emb(zcol, params["emb"])
    for c in range(NCONV):
        cp = cv[c]
        rn = _tc_rn(r, cp["node"]["w"], cp["node"]["b"][None, :])
        aggp = _sc_msg(rn, ehs[c], idxg, idxs)
        r = _tc_update(r, aggp, cp["upd1"]["w"], cp["upd1"]["b"][None, :],
                       cp["upd2"]["w"], cp["upd2"]["b"][None, :])

    ro = params["readout"]
    h = _tc_readout(r, ro["l1"]["w"], ro["l1"]["b"][None, :],
                    ro["l2"]["w"], ro["l2"]["b"][None, :])
    return h[:N]
